# baseline (speedup 1.0000x reference)
import functools

import jax
import jax.numpy as jnp
from jax import lax
from jax.experimental import pallas as pl
from jax.experimental.pallas import tpu as pltpu
from jax.experimental.pallas import tpu_sc as plsc

B = 8
N = 512 * 512
BINS = 32
NBINS2 = BINS * BINS

_info = plsc.get_sparse_core_info()
NC = _info.num_cores
NS = _info.num_subcores
NW = NC * NS
L = _info.num_lanes
ROWS = 512
COLS = 512
RPW = ROWS // NW
VPR = COLS // L


def _sc_hist_body(x_hbm, y_hbm, out_hbm, xv, yv, hist, semx, semy):
    wid = lax.axis_index("c") * NS + lax.axis_index("s")
    row0 = wid * RPW

    zeros = jnp.zeros((L,), jnp.float32)
    ones = jnp.ones((L,), jnp.float32)

    for zb in range(B):

        @plsc.parallel_loop(0, NBINS2 // L, unroll=8)
        def _zero(i):
            hist[zb, pl.ds(i * L, L)] = zeros

    cpx = pltpu.async_copy(x_hbm.at[0, 0, pl.ds(row0, RPW), :], xv.at[0], semx)
    cpy = pltpu.async_copy(y_hbm.at[0, 0, pl.ds(row0, RPW), :], yv.at[0], semy)
    for b in range(B):
        cur = b & 1
        cpx.wait()
        cpy.wait()
        if b + 1 < B:
            cpx = pltpu.async_copy(
                x_hbm.at[b + 1, 0, pl.ds(row0, RPW), :], xv.at[1 - cur], semx
            )
            cpy = pltpu.async_copy(
                y_hbm.at[b + 1, 0, pl.ds(row0, RPW), :], yv.at[1 - cur], semy
            )
        bvec = jnp.full((L,), b, jnp.int32)

        @plsc.parallel_loop(0, RPW * VPR, unroll=2)
        def _points(i):
            r = i // VPR
            c = i % VPR
            xr = xv[cur, r, pl.ds(c * L, L)]
            yr = yv[cur, r, pl.ds(c * L, L)]
            vx = xr * 16.0 + 16.0
            vy = yr * 16.0 + 16.0
            bx = lax.bitcast_convert_type(vx, jnp.uint32)
            by = lax.bitcast_convert_type(vy, jnp.uint32)
            valid = jnp.maximum(bx, by) <= jnp.uint32(0x42000000)
            tx = jnp.minimum(vx, 31.0).astype(jnp.int32)
            ty = jnp.minimum(vy, 31.0).astype(jnp.int32)
            flat = tx * BINS + ty
            plsc.addupdate_scatter(hist, [bvec, flat], ones, mask=valid)

    pltpu.sync_copy(hist, out_hbm.at[wid])


@functools.partial(
    pl.kernel,
    out_type=jax.ShapeDtypeStruct((NW, B, NBINS2), jnp.float32),
    mesh=plsc.VectorSubcoreMesh(core_axis_name="c", subcore_axis_name="s"),
    compiler_params=pltpu.CompilerParams(needs_layout_passes=False, use_tc_tiling_on_sc=True),
    scratch_types=[
        pltpu.VMEM((2, RPW, COLS), jnp.float32),
        pltpu.VMEM((2, RPW, COLS), jnp.float32),
        pltpu.VMEM((B, NBINS2), jnp.float32),
        pltpu.SemaphoreType.DMA,
        pltpu.SemaphoreType.DMA,
    ],
)
def _sc_hist(x_hbm, y_hbm, out_hbm, xv, yv, hist, semx, semy):
    _sc_hist_body(x_hbm, y_hbm, out_hbm, xv, yv, hist, semx, semy)


def _tc_mi_body(parts_ref, out_ref):
    h = jnp.sum(parts_ref[...], axis=0)
    k = lax.broadcasted_iota(jnp.int32, (NBINS2, BINS), 0)
    c = lax.broadcasted_iota(jnp.int32, (NBINS2, BINS), 1)
    row_sel = (k // BINS == c).astype(jnp.float32)
    col_sel = (k % BINS == c).astype(jnp.float32)
    px = jax.lax.dot(h, row_sel, precision=lax.Precision.HIGHEST)
    py = jax.lax.dot(h, col_sel, precision=lax.Precision.HIGHEST)
    tot = jnp.sum(h, axis=1, keepdims=True)
    hn = h / tot
    pxn = px / jnp.sum(px, axis=1, keepdims=True)
    pyn = py / jnp.sum(py, axis=1, keepdims=True)
    pxe = jax.lax.dot(pxn, row_sel.T, precision=lax.Precision.HIGHEST)
    pye = jax.lax.dot(pyn, col_sel.T, precision=lax.Precision.HIGHEST)
    pxy = pxe * pye
    mask = (hn > 0) & (pxe > 0) & (pye > 0)
    safe_ratio = jnp.where(mask, hn / jnp.where(mask, pxy, 1.0), 1.0)
    mi = jnp.sum(jnp.where(mask, hn * jnp.log(safe_ratio), 0.0))
    out_ref[0, 0] = -mi / B


def kernel(input, target):
    parts = _sc_hist(input, target)
    out = pl.pallas_call(
        _tc_mi_body,
        out_shape=jax.ShapeDtypeStruct((1, 1), jnp.float32),
        out_specs=pl.BlockSpec(memory_space=pltpu.SMEM),
    )(parts)
    return out[0, 0]


# speedup vs baseline: 1.1069x; 1.1069x over previous
import functools

import jax
import jax.numpy as jnp
from jax import lax
from jax.experimental import pallas as pl
from jax.experimental.pallas import tpu as pltpu
from jax.experimental.pallas import tpu_sc as plsc

B = 8
N = 512 * 512
BINS = 32
NBINS2 = BINS * BINS

_info = plsc.get_sparse_core_info()
NC = _info.num_cores
NS = _info.num_subcores
NW = NC * NS
L = _info.num_lanes
ROWS = 512
COLS = 512
RPW = ROWS // NW
VPR = COLS // L


def _sc_hist_body(x_hbm, y_hbm, out_hbm, xv, yv, hist, semx, semy):
    wid = lax.axis_index("c") * NS + lax.axis_index("s")
    row0 = wid * RPW

    zeros = jnp.zeros((L,), jnp.float32)
    ones = jnp.ones((L,), jnp.float32)

    for zb in range(B):

        @plsc.parallel_loop(0, NBINS2 // L, unroll=8)
        def _zero(i):
            hist[zb, i // 2, pl.ds((i % 2) * L, L)] = zeros

    cpx = pltpu.async_copy(x_hbm.at[0, 0, pl.ds(row0, RPW), :], xv.at[0], semx)
    cpy = pltpu.async_copy(y_hbm.at[0, 0, pl.ds(row0, RPW), :], yv.at[0], semy)
    for b in range(B):
        cur = b & 1
        cpx.wait()
        cpy.wait()
        if b + 1 < B:
            cpx = pltpu.async_copy(
                x_hbm.at[b + 1, 0, pl.ds(row0, RPW), :], xv.at[1 - cur], semx
            )
            cpy = pltpu.async_copy(
                y_hbm.at[b + 1, 0, pl.ds(row0, RPW), :], yv.at[1 - cur], semy
            )
        bvec = jnp.full((L,), b, jnp.int32)

        @plsc.parallel_loop(0, RPW * VPR, unroll=4)
        def _points(i):
            r = i // VPR
            c = i % VPR
            xr = xv[cur, r, pl.ds(c * L, L)]
            yr = yv[cur, r, pl.ds(c * L, L)]
            vx = xr * 16.0 + 16.0
            vy = yr * 16.0 + 16.0
            bx = lax.bitcast_convert_type(vx, jnp.uint32)
            by = lax.bitcast_convert_type(vy, jnp.uint32)
            valid = jnp.maximum(bx, by) <= jnp.uint32(0x42000000)
            tx = jnp.minimum(vx, 31.0).astype(jnp.int32)
            ty = jnp.minimum(vy, 31.0).astype(jnp.int32)
            plsc.addupdate_scatter(hist, [bvec, tx, ty], ones, mask=valid)

    pltpu.sync_copy(hist, out_hbm.at[wid])


@functools.partial(
    pl.kernel,
    out_type=jax.ShapeDtypeStruct((NW, B, BINS, BINS), jnp.float32),
    mesh=plsc.VectorSubcoreMesh(core_axis_name="c", subcore_axis_name="s"),
    compiler_params=pltpu.CompilerParams(needs_layout_passes=False, use_tc_tiling_on_sc=True),
    scratch_types=[
        pltpu.VMEM((2, RPW, COLS), jnp.float32),
        pltpu.VMEM((2, RPW, COLS), jnp.float32),
        pltpu.VMEM((B, BINS, BINS), jnp.float32),
        pltpu.SemaphoreType.DMA,
        pltpu.SemaphoreType.DMA,
    ],
)
def _sc_hist(x_hbm, y_hbm, out_hbm, xv, yv, hist, semx, semy):
    _sc_hist_body(x_hbm, y_hbm, out_hbm, xv, yv, hist, semx, semy)


def _tc_mi_body(parts_ref, out_ref):
    h = jnp.sum(parts_ref[...], axis=0)
    px = jnp.sum(h, axis=2, keepdims=True)
    py = jnp.sum(h, axis=1, keepdims=True)
    tot = jnp.sum(py, axis=2, keepdims=True)
    hn = h / tot
    pxn = px / tot
    pyn = py / tot
    pxy = pxn * pyn
    mask = (hn > 0) & (pxn > 0) & (pyn > 0)
    safe_ratio = jnp.where(mask, hn / jnp.where(mask, pxy, 1.0), 1.0)
    mi = jnp.sum(jnp.where(mask, hn * jnp.log(safe_ratio), 0.0))
    out_ref[0, 0] = -mi / B


def kernel(input, target):
    parts = _sc_hist(input, target)
    out = pl.pallas_call(
        _tc_mi_body,
        out_shape=jax.ShapeDtypeStruct((1, 1), jnp.float32),
        out_specs=pl.BlockSpec(memory_space=pltpu.SMEM),
    )(parts)
    return out[0, 0]
